# Initial kernel scaffold; baseline (speedup 1.0000x reference)
import jax
import jax.numpy as jnp
from jax.experimental import pallas as pl


def kernel(x, table):
    raise NotImplementedError("write your pallas kernel here")


# speedup vs baseline: 4.2371x; 4.2371x over previous
import functools
import math

import jax
import jax.numpy as jnp
from jax import lax
from jax.experimental import pallas as pl
from jax.experimental.pallas import tpu as pltpu
from jax.experimental.pallas import tpu_sc as plsc

D_MODEL = 256
VOCAB = 5
SCALE = math.sqrt(D_MODEL)

NC = 2
NS = 16
NW = NC * NS
LANES = 16
ROWS_PER_STEP = 16


def _make_kernel(B):
    b_per_w = B // NW
    n_steps = b_per_w // ROWS_PER_STEP
    mesh = plsc.VectorSubcoreMesh(core_axis_name="c", subcore_axis_name="s")

    @functools.partial(
        pl.kernel,
        mesh=mesh,
        out_type=jax.ShapeDtypeStruct((B, D_MODEL), jnp.float32),
        scratch_types=[
            pltpu.VMEM((VOCAB, D_MODEL), jnp.float32),
            pltpu.VMEM((b_per_w,), jnp.int32),
            pltpu.SMEM((b_per_w,), jnp.int32),
            pltpu.SemaphoreType.DMA,
        ],
    )
    def emb(table_hbm, idx_hbm, out_hbm, table_v, idx_v, idx_s, wsem):
        cid = lax.axis_index("c")
        sid = lax.axis_index("s")
        wid = sid * NC + cid
        base = wid * b_per_w

        pltpu.sync_copy(table_hbm, table_v)
        for r in range(VOCAB):
            for j in range(D_MODEL // LANES):
                sl = pl.ds(j * LANES, LANES)
                table_v[r, sl] = table_v[r, sl] * SCALE

        pltpu.sync_copy(idx_hbm.at[pl.ds(base, b_per_w)], idx_v)

        def step(i, _):
            i0 = i * ROWS_PER_STEP
            idx16 = idx_v[pl.ds(i0, ROWS_PER_STEP)]
            for k in range(ROWS_PER_STEP):
                r = idx16[k]
                pltpu.async_copy(
                    table_v.at[pl.ds(r, 1)],
                    out_hbm.at[pl.ds(base + i0 + k, 1)],
                    wsem,
                )
            return _

        lax.fori_loop(0, n_steps, step, 0, unroll=False)

        def drain(i, _):
            for k in range(ROWS_PER_STEP):
                pltpu.make_async_copy(
                    table_v.at[pl.ds(0, 1)],
                    out_hbm.at[pl.ds(base, 1)],
                    wsem,
                ).wait()
            return _

        lax.fori_loop(0, n_steps, drain, 0, unroll=False)

    return emb


def kernel(x, table):
    B0, B1 = x.shape
    B = B0 * B1
    idx = x.reshape(B).astype(jnp.int32)
    out = _make_kernel(B)(table, idx)
    return out.reshape(B0, B1, D_MODEL)
